# baseline (speedup 1.0000x reference)
import jax
import jax.numpy as jnp
from jax import lax
from jax.experimental import pallas as pl
from jax.experimental.pallas import tpu as pltpu
from jax.experimental.pallas import tpu_sc as plsc

_N = 10000
_E = 320000
_D = 128
_HD = _D // 2
_G = 64
_NPAD = 10240
_NC = 2
_NS = 16
_NW = _NC * _NS
_C = 80
_EW = _E // _NW
_MD = _EW // _C
_ET = _E // _NS
_M = _ET // _C
_RPT = _NPAD // _NS

_mesh = plsc.VectorSubcoreMesh(core_axis_name="c", subcore_axis_name="s")


def _deg_body(dst_hbm, zpad_hbm, out_hbm, acc, idxb, oneb, sem):
    cid = lax.axis_index("c")
    sid = lax.axis_index("s")
    wid = cid * _NS + sid
    pltpu.sync_copy(zpad_hbm.at[pl.ds(sid * _RPT, _RPT)],
                    acc.at[pl.ds(sid * _RPT, _RPT)])
    for j in range(_C // 16):
        oneb[pl.ds(j * 16, 16)] = jnp.ones((16,), jnp.float32)
    plsc.subcore_barrier()

    def step(m, carry):
        base = wid * _EW + m * _C
        pltpu.sync_copy(dst_hbm.at[pl.ds(base, _C)], idxb)
        pltpu.sync_copy(oneb, acc.at[idxb], add=True)
        return carry

    lax.fori_loop(0, _MD, step, 0)
    plsc.subcore_barrier()
    pltpu.sync_copy(acc.at[pl.ds(sid * _RPT, _RPT)],
                    out_hbm.at[pl.ds(cid * _NPAD + sid * _RPT, _RPT)])


def _sc_deg(dst, zpad):
    k = pl.kernel(
        _deg_body,
        out_type=jax.ShapeDtypeStruct((_NC * _NPAD,), jnp.float32),
        mesh=_mesh,
        scratch_types=[
            pltpu.VMEM_SHARED((_NPAD,), jnp.float32),
            pltpu.VMEM((_C,), jnp.int32),
            pltpu.VMEM((_C,), jnp.float32),
            pltpu.SemaphoreType.DMA,
        ],
    )
    return k(dst, zpad)


_NB = 4


def _scat_body(gsp_hbm, src_hbm, dst_hbm, zrows_hbm, out_hbm,
               gsh, acc, srcb, dstb, rows, isem, gsem, ssem):
    cid = lax.axis_index("c")
    sid = lax.axis_index("s")
    ebase = sid * _ET
    pltpu.sync_copy(gsp_hbm.at[cid, pl.ds(sid * _RPT, _RPT)],
                    gsh.at[pl.ds(sid * _RPT, _RPT)])
    pltpu.sync_copy(zrows_hbm.at[pl.ds(sid * _RPT, _RPT)],
                    acc.at[pl.ds(sid * _RPT, _RPT)])
    plsc.subcore_barrier()

    def I(m, j):
        base = ebase + m * _C
        pltpu.async_copy(src_hbm.at[pl.ds(base, _C)], srcb[j], isem[j])
        pltpu.async_copy(dst_hbm.at[pl.ds(base, _C)], dstb[j], isem[j])

    def Iw(j):
        pltpu.make_async_copy(src_hbm.at[pl.ds(0, _C)], srcb[j], isem[j]).wait()
        pltpu.make_async_copy(dst_hbm.at[pl.ds(0, _C)], dstb[j], isem[j]).wait()

    def G(j):
        pltpu.async_copy(gsh.at[srcb[j]], rows[j], gsem[j])

    def Gw(j):
        pltpu.make_async_copy(gsh.at[srcb[j]], rows[j], gsem[j]).wait()

    def S(j):
        pltpu.async_copy(rows[j], acc.at[dstb[j]], ssem[j], add=True)

    def Sw(j):
        pltpu.make_async_copy(rows[j], acc.at[dstb[j]], ssem[j]).wait()

    I(0, 0); I(1, 1)
    I(2, 2); Iw(0); G(0)
    I(3, 3); Iw(1); G(1); Gw(0); S(0)
    Sw(0); I(4, 0); Iw(2); G(2); Gw(1); S(1)
    Sw(1); I(5, 1); Iw(3); G(3); Gw(2); S(2)

    def step(t, carry):
        for j in range(_NB):
            m = _NB * t + j
            jn = (j + 2) % _NB
            jm = (j - 1) % _NB
            Sw(jn)
            base = ebase + (m + 2) * _C
            pltpu.async_copy(src_hbm.at[pl.ds(base, _C)], srcb[jn], isem[jn])
            pltpu.async_copy(dst_hbm.at[pl.ds(base, _C)], dstb[jn], isem[jn])
            Iw(j); G(j)
            Gw(jm); S(jm)
        return carry

    lax.fori_loop(1, _M // _NB, step, 0)
    Sw(2); Iw(0); G(0); Gw(3); S(3)
    Sw(3); Iw(1); G(1); Gw(0); S(0)
    Gw(1); S(1)
    Sw(0); Sw(1)

    plsc.subcore_barrier()
    pltpu.sync_copy(acc.at[pl.ds(sid * _RPT, _RPT)],
                    out_hbm.at[cid, pl.ds(sid * _RPT, _RPT)])


def _sc_scatter(gsp, src, dst, zrows):
    k = pl.kernel(
        _scat_body,
        out_type=jax.ShapeDtypeStruct((_NC, _NPAD, _HD), jnp.float32),
        mesh=_mesh,
        scratch_types=[
            pltpu.VMEM_SHARED((_NPAD, _HD), jnp.float32),
            pltpu.VMEM_SHARED((_NPAD, _HD), jnp.float32),
            [pltpu.VMEM((_C,), jnp.int32) for _ in range(_NB)],
            [pltpu.VMEM((_C,), jnp.int32) for _ in range(_NB)],
            [pltpu.VMEM((_C, _HD), jnp.float32) for _ in range(_NB)],
            [pltpu.SemaphoreType.DMA for _ in range(_NB)],
            [pltpu.SemaphoreType.DMA for _ in range(_NB)],
            [pltpu.SemaphoreType.DMA for _ in range(_NB)],
        ],
    )
    return k(gsp, src, dst, zrows)


def _store_split(gsp_o, g):
    gsp_o[0, : _N, :] = g[:, : _HD]
    gsp_o[1, : _N, :] = g[:, _HD:]
    zt = jnp.zeros((_NPAD - _N, _HD), jnp.float32)
    gsp_o[0, _N:, :] = zt
    gsp_o[1, _N:, :] = zt


def _merge_split(a2):
    return jnp.concatenate([a2[0, : _N, :], a2[1, : _N, :]], axis=-1)


def _enc_body(deg2, x, we1, be1, we2, be2, wc1, gsp_o, dinv_o):
    deg = deg2[0, :_N] + deg2[1, :_N] + 1.0
    dinv = lax.rsqrt(deg)
    dv = dinv[:, None]
    dinv_o[...] = dv
    h = jnp.maximum(x[...] @ we1[...] + be1[...][None, :], 0.0)
    h = h @ we2[...] + be2[...][None, :]
    _store_split(gsp_o, (h @ wc1[...]) * dv)


def _tc_encode(deg2, x, we1, be1, we2, be2, wc1):
    return pl.pallas_call(
        _enc_body,
        out_shape=(jax.ShapeDtypeStruct((_NC, _NPAD, _HD), jnp.float32),
                   jax.ShapeDtypeStruct((_N, 1), jnp.float32)),
    )(deg2, x, we1, be1, we2, be2, wc1)


def _conv_body(scat2, gsp, dinv, b, w, gn_o):
    s = _merge_split(scat2) + _merge_split(gsp)
    h = jnp.maximum(dinv[...] * s + b[...][None, :], 0.0)
    _store_split(gn_o, (h @ w[...]) * dinv[...])


def _tc_conv(scat2, gsp, dinv, b, w):
    return pl.pallas_call(
        _conv_body,
        out_shape=jax.ShapeDtypeStruct((_NC, _NPAD, _HD), jnp.float32),
    )(scat2, gsp, dinv, b, w)


def _tail_body(scat2, gsp, dinv, b, batch, wd1, bd1, wd2, bd2, out_o):
    s = _merge_split(scat2) + _merge_split(gsp)
    h = jnp.maximum(dinv[...] * s + b[...][None, :], 0.0)
    gid = lax.broadcasted_iota(jnp.int32, (_G, _N), 0)
    onehot = (batch[...][None, :] == gid).astype(jnp.float32)
    p = onehot @ h
    p = jnp.maximum(p @ wd1[...] + bd1[...][None, :], 0.0)
    out_o[...] = p @ wd2[...] + bd2[...][None, :]


def _tc_tail(scat2, gsp, dinv, b, batch, wd1, bd1, wd2, bd2):
    return pl.pallas_call(
        _tail_body,
        out_shape=jax.ShapeDtypeStruct((_G, _D), jnp.float32),
    )(scat2, gsp, dinv, b, batch, wd1, bd1, wd2, bd2)


def kernel(x, edge_index, batch, W_e1, b_e1, W_e2, b_e2,
           W_c1, b_c1, W_c2, b_c2, W_c3, b_c3,
           W_d1, b_d1, W_d2, b_d2):
    src = edge_index[0]
    dst = edge_index[1]
    zpad = jnp.zeros((_NPAD,), jnp.float32)
    zrows = jnp.zeros((_NPAD, _HD), jnp.float32)

    deg2 = _sc_deg(dst, zpad).reshape(_NC, _NPAD)
    g1, dinv = _tc_encode(deg2, x, W_e1, b_e1, W_e2, b_e2, W_c1)
    s1 = _sc_scatter(g1, src, dst, zrows)
    g2 = _tc_conv(s1, g1, dinv, b_c1, W_c2)
    s2 = _sc_scatter(g2, src, dst, zrows)
    g3 = _tc_conv(s2, g2, dinv, b_c2, W_c3)
    s3 = _sc_scatter(g3, src, dst, zrows)
    return _tc_tail(s3, g3, dinv, b_c3, batch, W_d1, b_d1, W_d2, b_d2)


# speedup vs baseline: 1.4225x; 1.4225x over previous
import functools

import jax
import jax.numpy as jnp
from jax import lax
from jax.experimental import pallas as pl
from jax.experimental.pallas import tpu as pltpu
from jax.experimental.pallas import tpu_sc as plsc

_N = 10000
_E = 320000
_D = 128
_G = 64
_NPAD = 10240
_NC = 2
_NS = 16
_NW = _NC * _NS
_EW = _E // _NW
_C = 80
_M = _EW // _C
_RPT = _NPAD // _NS
_DR = 80
_EPAD = _NW * _DR * 128

_mesh = plsc.VectorSubcoreMesh(core_axis_name="c", subcore_axis_name="s")


def _degv2_body(dstp_hbm, zpad_hbm, out_hbm, acc, dstb, oneb, sem):
    cid = lax.axis_index("c")
    sid = lax.axis_index("s")
    wid = cid * _NS + sid
    pltpu.sync_copy(zpad_hbm.at[pl.ds(sid * _RPT, _RPT)],
                    acc.at[pl.ds(sid * _RPT, _RPT)])
    for j in range(128 // 16):
        oneb[pl.ds(j * 16, 16)] = jnp.ones((16,), jnp.float32)
    pltpu.sync_copy(dstp_hbm.at[pl.ds(wid * _DR, _DR)], dstb)
    plsc.subcore_barrier()

    def S(r, j):
        pltpu.async_copy(oneb, acc.at[dstb.at[r]], sem[j], add=True)

    def Sw(j):
        pltpu.make_async_copy(oneb, acc.at[dstb.at[0]], sem[j]).wait()

    S(0, 0); S(1, 1); S(2, 2); S(3, 3)

    def step(t, carry):
        for j in range(4):
            Sw(j)
            S(4 * t + j, j)
        return carry

    lax.fori_loop(1, _DR // 4, step, 0)
    Sw(0); Sw(1); Sw(2); Sw(3)
    plsc.subcore_barrier()
    pltpu.sync_copy(acc.at[pl.ds(sid * _RPT, _RPT)],
                    out_hbm.at[pl.ds(cid * _NPAD + sid * _RPT, _RPT)])


def _sc_deg(dstp, zpad):
    k = pl.kernel(
        _degv2_body,
        out_type=jax.ShapeDtypeStruct((_NC * _NPAD,), jnp.float32),
        mesh=_mesh,
        scratch_types=[
            pltpu.VMEM_SHARED((_NPAD,), jnp.float32),
            pltpu.VMEM((_DR, 128), jnp.int32),
            pltpu.VMEM((128,), jnp.float32),
            [pltpu.SemaphoreType.DMA for _ in range(4)],
        ],
    )
    return k(dstp, zpad)


_NB = 4


_NI = 8


def _scat_body(g_hbm, src_hbm, dst_hbm, zrows_hbm, out_hbm,
               acc, srcb, dstb, rows, isem, gsem, ssem):
    cid = lax.axis_index("c")
    sid = lax.axis_index("s")
    wid = cid * _NS + sid
    ebase = wid * _EW
    pltpu.sync_copy(zrows_hbm, rows[0])
    for q in range(_RPT // _C):
        pltpu.sync_copy(rows[0], acc.at[pl.ds(sid * _RPT + q * _C, _C)])
    plsc.subcore_barrier()

    def I(m, j):
        base = ebase + m * _C
        pltpu.async_copy(src_hbm.at[pl.ds(base, _C)], srcb[j], isem[j])
        pltpu.async_copy(dst_hbm.at[pl.ds(base, _C)], dstb[j], isem[j])

    def Iw(j):
        pltpu.make_async_copy(src_hbm.at[pl.ds(0, _C)], srcb[j], isem[j]).wait()
        pltpu.make_async_copy(dst_hbm.at[pl.ds(0, _C)], dstb[j], isem[j]).wait()

    def G(ji, jr):
        pltpu.async_copy(g_hbm.at[srcb[ji]], rows[jr], gsem[jr])

    def Gw(ji, jr):
        pltpu.make_async_copy(g_hbm.at[srcb[ji]], rows[jr], gsem[jr]).wait()

    def S(ji, jr):
        pltpu.async_copy(rows[jr], acc.at[dstb[ji]], ssem[jr], add=True)

    def Sw(jr):
        pltpu.make_async_copy(rows[jr], acc.at[dstb[0]], ssem[jr]).wait()

    I(0, 0); I(1, 1); I(2, 2); I(3, 3); I(4, 4)
    Iw(0); G(0, 0)
    I(5, 5); Iw(1); G(1, 1); Gw(0, 0); S(0, 0)
    I(6, 6); Iw(2); G(2, 2); Gw(1, 1); S(1, 1)
    I(7, 7); Iw(3); G(3, 3); Gw(2, 2); S(2, 2)
    Sw(0); I(8, 0); Iw(4); G(4, 0); Gw(3, 3); S(3, 3)
    Sw(1); I(9, 1); Iw(5); G(5, 1); Gw(4, 0); S(4, 0)
    Sw(2); I(10, 2); Iw(6); G(6, 2); Gw(5, 1); S(5, 1)
    Sw(3); I(11, 3); Iw(7); G(7, 3); Gw(6, 2); S(6, 2)
    Sw(0); I(12, 4); Iw(0); G(0, 0); Gw(7, 3); S(7, 3)

    def step(t, carry):
        for j in range(_NI):
            m = _NI * t + j
            jr = (j + 1) % _NB
            jm = j % _NB
            Sw(jr)
            base = ebase + (m + 5) * _C
            jn = (j + 5) % _NI
            pltpu.async_copy(src_hbm.at[pl.ds(base, _C)], srcb[jn], isem[jn])
            pltpu.async_copy(dst_hbm.at[pl.ds(base, _C)], dstb[jn], isem[jn])
            Iw((j + 1) % _NI); G((j + 1) % _NI, jr)
            Gw(j, jm); S(j, jm)
        return carry

    lax.fori_loop(1, 15, step, 0)
    Sw(1); Iw(1); G(1, 1); Gw(0, 0); S(0, 0)
    Sw(2); Iw(2); G(2, 2); Gw(1, 1); S(1, 1)
    Sw(3); Iw(3); G(3, 3); Gw(2, 2); S(2, 2)
    Sw(0); Iw(4); G(4, 0); Gw(3, 3); S(3, 3)
    Sw(1); Gw(4, 0); S(4, 0)
    Sw(2); Sw(3); Sw(0)

    plsc.subcore_barrier()
    pltpu.sync_copy(acc.at[pl.ds(sid * _RPT, _RPT)],
                    out_hbm.at[cid, pl.ds(sid * _RPT, _RPT)])


def _sc_scatter(g, src, dst, zrows):
    k = pl.kernel(
        _scat_body,
        out_type=jax.ShapeDtypeStruct((_NC, _NPAD, _D), jnp.float32),
        mesh=_mesh,
        scratch_types=[
            pltpu.VMEM_SHARED((_NPAD, _D), jnp.float32),
            [pltpu.VMEM((_C,), jnp.int32) for _ in range(_NI)],
            [pltpu.VMEM((_C,), jnp.int32) for _ in range(_NI)],
            [pltpu.VMEM((_C, _D), jnp.float32) for _ in range(_NB)],
            [pltpu.SemaphoreType.DMA for _ in range(_NI)],
            [pltpu.SemaphoreType.DMA for _ in range(_NB)],
            [pltpu.SemaphoreType.DMA for _ in range(_NB)],
        ],
    )
    return k(g, src, dst, zrows)


def _enc_body(deg2, x, we1, be1, we2, be2, wc1, g1_o, dinv_o):
    deg = deg2[0, :_N] + deg2[1, :_N] + 1.0
    dinv = lax.rsqrt(deg)
    dv = dinv[:, None]
    dinv_o[...] = dv
    h = jnp.maximum(x[...] @ we1[...] + be1[...][None, :], 0.0)
    h = h @ we2[...] + be2[...][None, :]
    g1_o[...] = (h @ wc1[...]) * dv


def _tc_encode(deg2, x, we1, be1, we2, be2, wc1):
    return pl.pallas_call(
        _enc_body,
        out_shape=(jax.ShapeDtypeStruct((_N, _D), jnp.float32),
                   jax.ShapeDtypeStruct((_N, 1), jnp.float32)),
    )(deg2, x, we1, be1, we2, be2, wc1)


def _conv_body(scat2, g, dinv, b, w, gn_o):
    s = scat2[0, :_N] + scat2[1, :_N] + g[...]
    h = jnp.maximum(dinv[...] * s + b[...][None, :], 0.0)
    gn_o[...] = (h @ w[...]) * dinv[...]


def _tc_conv(scat2, g, dinv, b, w):
    return pl.pallas_call(
        _conv_body,
        out_shape=jax.ShapeDtypeStruct((_N, _D), jnp.float32),
    )(scat2, g, dinv, b, w)


def _tail_body(scat2, g, dinv, b, batch, wd1, bd1, wd2, bd2, out_o):
    s = scat2[0, :_N] + scat2[1, :_N] + g[...]
    h = jnp.maximum(dinv[...] * s + b[...][None, :], 0.0)
    gid = lax.broadcasted_iota(jnp.int32, (_G, _N), 0)
    onehot = (batch[...][None, :] == gid).astype(jnp.float32)
    p = onehot @ h
    p = jnp.maximum(p @ wd1[...] + bd1[...][None, :], 0.0)
    out_o[...] = p @ wd2[...] + bd2[...][None, :]


def _tc_tail(scat2, g, dinv, b, batch, wd1, bd1, wd2, bd2):
    return pl.pallas_call(
        _tail_body,
        out_shape=jax.ShapeDtypeStruct((_G, _D), jnp.float32),
    )(scat2, g, dinv, b, batch, wd1, bd1, wd2, bd2)


def kernel(x, edge_index, batch, W_e1, b_e1, W_e2, b_e2,
           W_c1, b_c1, W_c2, b_c2, W_c3, b_c3,
           W_d1, b_d1, W_d2, b_d2):
    src = edge_index[0]
    dst = edge_index[1]
    zpad = jnp.zeros((_NPAD,), jnp.float32)
    zrows = jnp.zeros((_C, _D), jnp.float32)
    dstp = jnp.concatenate(
        [dst, jnp.full((_EPAD - _E,), _N, jnp.int32)]).reshape(-1, 128)

    deg2 = _sc_deg(dstp, zpad).reshape(_NC, _NPAD)
    g1, dinv = _tc_encode(deg2, x, W_e1, b_e1, W_e2, b_e2, W_c1)
    s1 = _sc_scatter(g1, src, dst, zrows)
    g2 = _tc_conv(s1, g1, dinv, b_c1, W_c2)
    s2 = _sc_scatter(g2, src, dst, zrows)
    g3 = _tc_conv(s2, g2, dinv, b_c2, W_c3)
    s3 = _sc_scatter(g3, src, dst, zrows)
    return _tc_tail(s3, g3, dinv, b_c3, batch, W_d1, b_d1, W_d2, b_d2)


# speedup vs baseline: 1.5577x; 1.0950x over previous
import functools

import jax
import jax.numpy as jnp
from jax import lax
from jax.experimental import pallas as pl
from jax.experimental.pallas import tpu as pltpu
from jax.experimental.pallas import tpu_sc as plsc

_N = 10000
_E = 320000
_D = 128
_G = 64
_NPAD = 10240
_NC = 2
_NS = 16
_NW = _NC * _NS
_EW = _E // _NW
_C = 80
_M = _EW // _C
_RPT = _NPAD // _NS
_DR = 80
_EPAD = _NW * _DR * 128

_mesh = plsc.VectorSubcoreMesh(core_axis_name="c", subcore_axis_name="s")


def _degv2_body(dstp_hbm, zpad_hbm, out_hbm, acc, dstb, oneb, sem):
    cid = lax.axis_index("c")
    sid = lax.axis_index("s")
    wid = cid * _NS + sid
    pltpu.sync_copy(zpad_hbm.at[pl.ds(sid * _RPT, _RPT)],
                    acc.at[pl.ds(sid * _RPT, _RPT)])
    for j in range(128 // 16):
        oneb[pl.ds(j * 16, 16)] = jnp.ones((16,), jnp.float32)
    pltpu.sync_copy(dstp_hbm.at[pl.ds(wid * _DR, _DR)], dstb)
    plsc.subcore_barrier()

    def S(r, j):
        pltpu.async_copy(oneb, acc.at[dstb.at[r]], sem[j], add=True)

    def Sw(j):
        pltpu.make_async_copy(oneb, acc.at[dstb.at[0]], sem[j]).wait()

    S(0, 0); S(1, 1); S(2, 2); S(3, 3)

    def step(t, carry):
        for j in range(4):
            Sw(j)
            S(4 * t + j, j)
        return carry

    lax.fori_loop(1, _DR // 4, step, 0)
    Sw(0); Sw(1); Sw(2); Sw(3)
    plsc.subcore_barrier()
    pltpu.sync_copy(acc.at[pl.ds(sid * _RPT, _RPT)],
                    out_hbm.at[pl.ds(cid * _NPAD + sid * _RPT, _RPT)])


def _sc_deg(dstp, zpad):
    k = pl.kernel(
        _degv2_body,
        out_type=jax.ShapeDtypeStruct((_NC * _NPAD,), jnp.float32),
        mesh=_mesh,
        scratch_types=[
            pltpu.VMEM_SHARED((_NPAD,), jnp.float32),
            pltpu.VMEM((_DR, 128), jnp.int32),
            pltpu.VMEM((128,), jnp.float32),
            [pltpu.SemaphoreType.DMA for _ in range(4)],
        ],
    )
    return k(dstp, zpad)


_NB = 4


_NI = 8


def _scat_body(g_hbm, src_hbm, dst_hbm, zrows_hbm, out_hbm,
               acc, srcb, dstb, rows, isem, gsem, ssem):
    cid = lax.axis_index("c")
    sid = lax.axis_index("s")
    wid = cid * _NS + sid
    ebase = wid * _EW
    pltpu.sync_copy(zrows_hbm, rows[0])
    for q in range(_RPT // _C):
        pltpu.sync_copy(rows[0], acc.at[pl.ds(sid * _RPT + q * _C, _C)])
    plsc.subcore_barrier()

    def I(m, j):
        base = ebase + m * _C
        pltpu.async_copy(src_hbm.at[pl.ds(base, _C)], srcb[j], isem[j])
        pltpu.async_copy(dst_hbm.at[pl.ds(base, _C)], dstb[j], isem[j])

    def Iw(j):
        pltpu.make_async_copy(src_hbm.at[pl.ds(0, _C)], srcb[j], isem[j]).wait()
        pltpu.make_async_copy(dst_hbm.at[pl.ds(0, _C)], dstb[j], isem[j]).wait()

    def G(ji, jr):
        pltpu.async_copy(g_hbm.at[srcb[ji]], rows[jr], gsem[jr])

    def Gw(ji, jr):
        pltpu.make_async_copy(g_hbm.at[srcb[ji]], rows[jr], gsem[jr]).wait()

    def S(ji, jr):
        pltpu.async_copy(rows[jr], acc.at[dstb[ji]], ssem[jr], add=True)

    def Sw(jr):
        pltpu.make_async_copy(rows[jr], acc.at[dstb[0]], ssem[jr]).wait()

    I(0, 0); I(1, 1); I(2, 2); I(3, 3); I(4, 4)
    Iw(0); G(0, 0); Iw(1); G(1, 1)
    I(5, 5); Iw(2); G(2, 2); Gw(0, 0); S(0, 0)
    I(6, 6); Iw(3); G(3, 3); Gw(1, 1); S(1, 1)
    Sw(0); I(7, 7); Iw(4); G(4, 0); Gw(2, 2); S(2, 2)
    Sw(1); I(8, 0); Iw(5); G(5, 1); Gw(3, 3); S(3, 3)
    Sw(2); I(9, 1); Iw(6); G(6, 2); Gw(4, 0); S(4, 0)
    Sw(3); I(10, 2); Iw(7); G(7, 3); Gw(5, 1); S(5, 1)
    Sw(0); I(11, 3); Iw(0); G(0, 0); Gw(6, 2); S(6, 2)
    Sw(1); I(12, 4); Iw(1); G(1, 1); Gw(7, 3); S(7, 3)

    def step(t, carry):
        for j in range(_NI):
            m = _NI * t + j
            jg = (j + 2) % _NI
            jr = (j + 2) % _NB
            jm = j % _NB
            Sw(jr)
            base = ebase + (m + 5) * _C
            jn = (j + 5) % _NI
            pltpu.async_copy(src_hbm.at[pl.ds(base, _C)], srcb[jn], isem[jn])
            pltpu.async_copy(dst_hbm.at[pl.ds(base, _C)], dstb[jn], isem[jn])
            Iw(jg); G(jg, jr)
            Gw(j, jm); S(j, jm)
        return carry

    lax.fori_loop(1, 15, step, 0)
    Sw(2); Iw(2); G(2, 2); Gw(0, 0); S(0, 0)
    Sw(3); Iw(3); G(3, 3); Gw(1, 1); S(1, 1)
    Sw(0); Iw(4); G(4, 0); Gw(2, 2); S(2, 2)
    Sw(1); Gw(3, 3); S(3, 3)
    Sw(2); Gw(4, 0); S(4, 0)
    Sw(3); Sw(0)

    plsc.subcore_barrier()
    pltpu.sync_copy(acc.at[pl.ds(sid * _RPT, _RPT)],
                    out_hbm.at[cid, pl.ds(sid * _RPT, _RPT)])


def _sc_scatter(g, src, dst, zrows):
    k = pl.kernel(
        _scat_body,
        out_type=jax.ShapeDtypeStruct((_NC, _NPAD, _D), jnp.float32),
        mesh=_mesh,
        scratch_types=[
            pltpu.VMEM_SHARED((_NPAD, _D), jnp.float32),
            [pltpu.VMEM((_C,), jnp.int32) for _ in range(_NI)],
            [pltpu.VMEM((_C,), jnp.int32) for _ in range(_NI)],
            [pltpu.VMEM((_C, _D), jnp.float32) for _ in range(_NB)],
            [pltpu.SemaphoreType.DMA for _ in range(_NI)],
            [pltpu.SemaphoreType.DMA for _ in range(_NB)],
            [pltpu.SemaphoreType.DMA for _ in range(_NB)],
        ],
    )
    return k(g, src, dst, zrows)


def _enc_body(deg2, x, we1, be1, we2, be2, wc1, g1_o, dinv_o):
    deg = deg2[0, :_N] + deg2[1, :_N] + 1.0
    dinv = lax.rsqrt(deg)
    dv = dinv[:, None]
    dinv_o[...] = dv
    h = jnp.maximum(x[...] @ we1[...] + be1[...][None, :], 0.0)
    h = h @ we2[...] + be2[...][None, :]
    g1_o[...] = (h @ wc1[...]) * dv


def _tc_encode(deg2, x, we1, be1, we2, be2, wc1):
    return pl.pallas_call(
        _enc_body,
        out_shape=(jax.ShapeDtypeStruct((_N, _D), jnp.float32),
                   jax.ShapeDtypeStruct((_N, 1), jnp.float32)),
    )(deg2, x, we1, be1, we2, be2, wc1)


def _conv_body(scat2, g, dinv, b, w, gn_o):
    s = scat2[0, :_N] + scat2[1, :_N] + g[...]
    h = jnp.maximum(dinv[...] * s + b[...][None, :], 0.0)
    gn_o[...] = (h @ w[...]) * dinv[...]


def _tc_conv(scat2, g, dinv, b, w):
    return pl.pallas_call(
        _conv_body,
        out_shape=jax.ShapeDtypeStruct((_N, _D), jnp.float32),
    )(scat2, g, dinv, b, w)


def _tail_body(scat2, g, dinv, b, batch, wd1, bd1, wd2, bd2, out_o):
    s = scat2[0, :_N] + scat2[1, :_N] + g[...]
    h = jnp.maximum(dinv[...] * s + b[...][None, :], 0.0)
    gid = lax.broadcasted_iota(jnp.int32, (_G, _N), 0)
    onehot = (batch[...][None, :] == gid).astype(jnp.float32)
    p = onehot @ h
    p = jnp.maximum(p @ wd1[...] + bd1[...][None, :], 0.0)
    out_o[...] = p @ wd2[...] + bd2[...][None, :]


def _tc_tail(scat2, g, dinv, b, batch, wd1, bd1, wd2, bd2):
    return pl.pallas_call(
        _tail_body,
        out_shape=jax.ShapeDtypeStruct((_G, _D), jnp.float32),
    )(scat2, g, dinv, b, batch, wd1, bd1, wd2, bd2)


def kernel(x, edge_index, batch, W_e1, b_e1, W_e2, b_e2,
           W_c1, b_c1, W_c2, b_c2, W_c3, b_c3,
           W_d1, b_d1, W_d2, b_d2):
    src = edge_index[0]
    dst = edge_index[1]
    zpad = jnp.zeros((_NPAD,), jnp.float32)
    zrows = jnp.zeros((_C, _D), jnp.float32)
    dstp = jnp.concatenate(
        [dst, jnp.full((_EPAD - _E,), _N, jnp.int32)]).reshape(-1, 128)

    deg2 = _sc_deg(dstp, zpad).reshape(_NC, _NPAD)
    g1, dinv = _tc_encode(deg2, x, W_e1, b_e1, W_e2, b_e2, W_c1)
    s1 = _sc_scatter(g1, src, dst, zrows)
    g2 = _tc_conv(s1, g1, dinv, b_c1, W_c2)
    s2 = _sc_scatter(g2, src, dst, zrows)
    g3 = _tc_conv(s2, g2, dinv, b_c2, W_c3)
    s3 = _sc_scatter(g3, src, dst, zrows)
    return _tc_tail(s3, g3, dinv, b_c3, batch, W_d1, b_d1, W_d2, b_d2)
